# baseline (device time: 99882 ns/iter reference)
import jax
import jax.numpy as jnp
from jax import lax
from jax.experimental import pallas as pl
from jax.experimental.pallas import tpu as pltpu

N_DEV = 8
SQ = 1024
D = 1024
HQ = 8
DH = 128
BLK = SQ // N_DEV
BAND = 384
WIN = 128
HALO = 128
SCALE = 0.08838834764831843

_FWD = {
    1: [(3, 2), (4, 5)],
    3: [(4, 7)],
    4: [(2, 6)],
}
_PROD_ORDER = [1, 2, 3, 4, 5, 6, 7, 0]


def kernel(x, Wq, K_ext, V_ext, Wo):
    x2 = x.reshape(SQ, D)
    k2 = K_ext.reshape(K_ext.shape[1], HQ * DH)
    v2 = V_ext.reshape(V_ext.shape[1], HQ * DH)

    def body(x_ref, wq_ref, k_ref, v_ref, wo_ref, out_ref,
             q_scr, ctx_scr, ctx_slice, khalo, vhalo, agst,
             halo_send, halo_recv, scat_send, scat_recv,
             asend, arecv):
        pos = lax.axis_index("i")

        @pl.when(pos == 1)
        def _():
            k_rdma = pltpu.make_async_remote_copy(
                src_ref=k_ref.at[pl.ds(0, HALO), :], dst_ref=khalo,
                send_sem=halo_send.at[0], recv_sem=halo_recv.at[0],
                device_id=(0,), device_id_type=pl.DeviceIdType.MESH)
            v_rdma = pltpu.make_async_remote_copy(
                src_ref=v_ref.at[pl.ds(0, HALO), :], dst_ref=vhalo,
                send_sem=halo_send.at[1], recv_sem=halo_recv.at[1],
                device_id=(0,), device_id_type=pl.DeviceIdType.MESH)
            k_rdma.start()
            v_rdma.start()
            k_rdma.wait_send()
            v_rdma.wait_send()

        @pl.when(pos == 0)
        def _():
            k_wait = pltpu.make_async_remote_copy(
                src_ref=k_ref.at[pl.ds(0, HALO), :], dst_ref=khalo,
                send_sem=halo_send.at[0], recv_sem=halo_recv.at[0],
                device_id=(1,), device_id_type=pl.DeviceIdType.MESH)
            v_wait = pltpu.make_async_remote_copy(
                src_ref=v_ref.at[pl.ds(0, HALO), :], dst_ref=vhalo,
                send_sem=halo_send.at[1], recv_sem=halo_recv.at[1],
                device_id=(1,), device_id_type=pl.DeviceIdType.MESH)

            q_scr[...] = jnp.dot(x_ref[...], wq_ref[...],
                                 preferred_element_type=jnp.float32)

            r_i = lax.broadcasted_iota(jnp.int32, (BLK, BAND), 0)
            c_i = lax.broadcasted_iota(jnp.int32, (BLK, BAND), 1)
            mask0 = jnp.abs(r_i - c_i) <= WIN
            maskn = jnp.abs(r_i - c_i + WIN) <= WIN

            rdmas = []
            for qb in _PROD_ORDER:
                s = max(0, BLK * qb - WIN)
                q_blk = q_scr[pl.ds(qb * BLK, BLK), :]
                if qb < N_DEV - 1:
                    kband = k_ref[pl.ds(s, BAND), :]
                    vband = v_ref[pl.ds(s, BAND), :]
                else:
                    k_wait.wait_recv()
                    v_wait.wait_recv()
                    kband = jnp.concatenate(
                        [k_ref[pl.ds(s, BAND - HALO), :], khalo[...]], axis=0)
                    vband = jnp.concatenate(
                        [v_ref[pl.ds(s, BAND - HALO), :], vhalo[...]], axis=0)
                mask = mask0 if qb == 0 else maskn
                for h in range(HQ):
                    qh = q_blk[:, h * DH:(h + 1) * DH]
                    kb = kband[:, h * DH:(h + 1) * DH]
                    vb = vband[:, h * DH:(h + 1) * DH]
                    scores = lax.dot_general(
                        qh, kb, (((1,), (1,)), ((), ())),
                        preferred_element_type=jnp.float32) * SCALE
                    scores = jnp.where(mask, scores, -1e9)
                    w = jnp.exp(scores)
                    l = jnp.sum(w, axis=1, keepdims=True)
                    ctx_h = jnp.dot(w, vb, preferred_element_type=jnp.float32)
                    ctx_scr[pl.ds(qb * BLK, BLK), pl.ds(h * DH, DH)] = (
                        ctx_h / l)
                if qb != 0:
                    r = pltpu.make_async_remote_copy(
                        src_ref=ctx_scr.at[pl.ds(qb * BLK, BLK), :],
                        dst_ref=ctx_slice,
                        send_sem=scat_send.at[qb - 1], recv_sem=scat_recv,
                        device_id=(qb,), device_id_type=pl.DeviceIdType.MESH)
                    r.start()
                    rdmas.append(r)
            ctx_slice[...] = ctx_scr[pl.ds(0, BLK), :]
            for r in rdmas:
                r.wait_send()

        @pl.when(pos != 0)
        def _():
            scat_wait = pltpu.make_async_remote_copy(
                src_ref=ctx_slice, dst_ref=ctx_slice,
                send_sem=scat_send.at[0], recv_sem=scat_recv,
                device_id=(0,), device_id_type=pl.DeviceIdType.MESH)
            scat_wait.wait_recv()

        out_slice = jnp.dot(ctx_slice[...], wo_ref[...],
                            preferred_element_type=jnp.float32)
        out_ref[pl.ds(pos * BLK, BLK), :] = out_slice
        agst[0] = out_slice

        for rr in range(N_DEV):
            @pl.when(pos == rr)
            def _(rr=rr):
                started = []
                for si, j in enumerate((1, 3, 4)):
                    r = pltpu.make_async_remote_copy(
                        src_ref=agst.at[0], dst_ref=agst.at[j],
                        send_sem=asend.at[si], recv_sem=arecv.at[j],
                        device_id=(rr ^ j,),
                        device_id_type=pl.DeviceIdType.MESH)
                    r.start()
                    started.append(r)
                fsi = 3
                for o in _PROD_ORDER:
                    if o == rr:
                        continue
                    j = o ^ rr
                    wait = pltpu.make_async_remote_copy(
                        src_ref=agst.at[0], dst_ref=agst.at[j],
                        send_sem=asend.at[0], recv_sem=arecv.at[j],
                        device_id=(rr,), device_id_type=pl.DeviceIdType.MESH)
                    wait.wait_recv()
                    for t_xor, dslot in _FWD.get(j, ()):
                        r = pltpu.make_async_remote_copy(
                            src_ref=agst.at[j], dst_ref=agst.at[dslot],
                            send_sem=asend.at[fsi], recv_sem=arecv.at[dslot],
                            device_id=(rr ^ t_xor,),
                            device_id_type=pl.DeviceIdType.MESH)
                        r.start()
                        started.append(r)
                        fsi += 1
                    out_ref[pl.ds(o * BLK, BLK), :] = agst[j]
                for r in started:
                    r.wait_send()

    out = pl.pallas_call(
        body,
        out_shape=jax.ShapeDtypeStruct((SQ, D), jnp.float32),
        in_specs=[pl.BlockSpec(memory_space=pltpu.VMEM)] * 5,
        out_specs=pl.BlockSpec(memory_space=pltpu.VMEM),
        scratch_shapes=[
            pltpu.VMEM((SQ, D), jnp.float32),
            pltpu.VMEM((SQ, D), jnp.float32),
            pltpu.VMEM((BLK, D), jnp.float32),
            pltpu.VMEM((HALO, HQ * DH), jnp.float32),
            pltpu.VMEM((HALO, HQ * DH), jnp.float32),
            pltpu.VMEM((N_DEV, BLK, D), jnp.float32),
            pltpu.SemaphoreType.DMA((2,)),
            pltpu.SemaphoreType.DMA((2,)),
            pltpu.SemaphoreType.DMA((N_DEV - 1,)),
            pltpu.SemaphoreType.DMA,
            pltpu.SemaphoreType.DMA((7,)),
            pltpu.SemaphoreType.DMA((N_DEV,)),
        ],
    )(x2, Wq, k2, v2, Wo)
    return out.reshape(1, SQ, D)
